# baseline (speedup 1.0000x reference)
import functools

import jax
import jax.numpy as jnp
from jax import lax
from jax.experimental import pallas as pl
from jax.experimental.pallas import tpu as pltpu
from jax.experimental.pallas import tpu_sc as plsc

N = 10000
E = 320000
D = 128

NC = 2
NS = 16
NW = NC * NS
L = 16

NN = 312
LASTN = N - (NW - 1) * NN
NNMAX = max(NN, LASTN)
WIN = 352
CH = 128
KSUB = D // L
HD = D // 2
GSUB = HD // L
HMASK = -65536
BS_IT = 9
TROWS = 640
TLAST = N - (NS - 1) * TROWS


def _prologue_body(feat_ref, deg_ref, packed_ref, rsd_ref):
    rs = lax.rsqrt(deg_ref[...])
    rsd_ref[...] = rs
    scaled = feat_ref[...] * rs[:, None]
    sbf = scaled.astype(jnp.bfloat16)
    lo = lax.bitcast_convert_type(sbf[:, :HD], jnp.uint16).astype(jnp.int32)
    hi = lax.bitcast_convert_type(sbf[:, HD:], jnp.uint16).astype(jnp.int32)
    packed_ref[...] = lo | (hi << 16)


def _epilogue_body(agg_ref, rsd_ref, out_ref):
    out_ref[...] = agg_ref[...] * rsd_ref[...]


def _rowwise_call(body, packed_out):
    del packed_out
    return pl.pallas_call(
        body,
        out_shape=[
            jax.ShapeDtypeStruct((N, HD), jnp.int32),
            jax.ShapeDtypeStruct((N,), jnp.float32),
        ],
    )


def _sc_body(rp_hbm, col_hbm, feat_hbm, rsd_hbm, out_hbm,
             idx_buf, row_buf, out_stage, rp_v, rp_s, rsd_v, rsd_s, tab_sh,
             sem_idx, sem_row, sem_tab):
    cid = lax.axis_index("c")
    sid = lax.axis_index("s")
    wid = cid * NS + sid
    n_lo = wid * NN
    nn = jnp.where(wid == NW - 1, LASTN, NN).astype(jnp.int32)
    lo8 = (n_lo // 8) * 8
    off = (n_lo - lo8).astype(jnp.int32)

    t_lo = sid * TROWS

    def tab_copy(rows):
        return pltpu.make_async_copy(
            feat_hbm.at[pl.ds(t_lo, rows)], tab_sh.at[pl.ds(t_lo, rows)],
            sem_tab)

    @pl.when(sid < NS - 1)
    def _():
        tab_copy(TROWS).start()

    @pl.when(sid == NS - 1)
    def _():
        tab_copy(TLAST).start()

    pltpu.sync_copy(rp_hbm.at[pl.ds(lo8, WIN)], rp_v)
    pltpu.sync_copy(rsd_hbm.at[pl.ds(lo8, WIN)], rsd_v)
    for q in range(WIN // L):
        iv = rp_v[pl.ds(q * L, L)]
        fv = rsd_v[pl.ds(q * L, L)]
        for t in range(L):
            rp_s[q * L + t] = iv[t]
            rsd_s[q * L + t] = fv[t]

    zeros = jnp.zeros((L,), jnp.float32)

    def zbody(i, carry):
        for k in range(KSUB):
            out_stage[i, pl.ds(k * L, L)] = zeros
        return carry

    lax.fori_loop(0, NNMAX, zbody, 0)

    e0 = rp_s[off]
    eN = rp_s[off + nn]
    a0 = (e0 // 8) * 8
    nch = (eN - a0 + CH - 1) // CH

    def bsearch(base, val, hi0):
        def bb(_, s):
            lo, hi = s
            mid = (lo + hi) // 2
            v = rp_s[base + mid]
            big = v >= val
            return (jnp.where(big, lo, mid + 1), jnp.where(big, mid, hi))
        lo, _ = lax.fori_loop(0, BS_IT, bb, (jnp.int32(0), hi0))
        return lo

    def cbase(c):
        return jnp.minimum(a0 + c * CH, E - CH)

    def start_idx(c):
        pltpu.make_async_copy(
            col_hbm.at[pl.ds(cbase(c), CH)],
            idx_buf.at[c % 2], sem_idx.at[c % 2]).start()

    def wait_idx(c):
        pltpu.make_async_copy(
            col_hbm.at[pl.ds(cbase(c), CH)],
            idx_buf.at[c % 2], sem_idx.at[c % 2]).wait()

    def start_row(c):
        pltpu.make_async_copy(
            tab_sh.at[idx_buf.at[c % 2]],
            row_buf.at[c % 2], sem_row.at[c % 2]).start()

    def wait_row(c):
        pltpu.make_async_copy(
            tab_sh.at[idx_buf.at[c % 2]],
            row_buf.at[c % 2], sem_row.at[c % 2]).wait()

    @pl.when(nch > 0)
    def _():
        start_idx(0)

    @pl.when(nch > 1)
    def _():
        start_idx(1)

    @pl.when(sid < NS - 1)
    def _():
        tab_copy(TROWS).wait()

    @pl.when(sid == NS - 1)
    def _():
        tab_copy(TLAST).wait()

    plsc.subcore_barrier()

    @pl.when(nch > 0)
    def _():
        wait_idx(0)
        start_row(0)

    def chunk_body(c, carry):
        chunk_lo = a0 + c * CH
        chunk_hi = chunk_lo + CH
        base = cbase(c)
        wait_row(c)

        @pl.when(c + 2 < nch)
        def _():
            start_idx(c + 2)

        @pl.when(c + 1 < nch)
        def _():
            wait_idx(c + 1)
            start_row(c + 1)

        i_begin = bsearch(off + 1, chunk_lo + 1, nn)
        i_end = bsearch(off, chunk_hi, nn)

        def node_body(i, carry2):
            start = rp_s[off + i]
            end = rp_s[off + i + 1]
            a = jnp.maximum(start, chunk_lo)
            b = jnp.minimum(end, chunk_hi)
            accs = tuple(out_stage[i, pl.ds(k * L, L)] for k in range(KSUB))

            rb = c % 2

            def add_row(r, accs_):
                na = list(accs_)
                for g in range(GSUB):
                    w = row_buf[rb, r, pl.ds(g * L, L)]
                    lo = lax.bitcast_convert_type(w << 16, jnp.float32)
                    hi = lax.bitcast_convert_type(w & HMASK, jnp.float32)
                    na[g] = na[g] + lo
                    na[GSUB + g] = na[GSUB + g] + hi
                return tuple(na)

            r0 = a - base
            nquad = (b - a) // 4

            def ebody4(p, accs_):
                r = r0 + 4 * p
                return add_row(
                    r + 3, add_row(r + 2, add_row(r + 1, add_row(r, accs_))))

            accs = lax.fori_loop(0, nquad, ebody4, accs)

            def etail(j, accs_):
                return add_row(j - base, accs_)

            accs = lax.fori_loop(a + 4 * nquad, b, etail, accs)
            m = jnp.where(end <= chunk_hi, rsd_s[off + i], 1.0)
            for k in range(KSUB):
                out_stage[i, pl.ds(k * L, L)] = accs[k] * m
            return carry2

        lax.fori_loop(i_begin, i_end, node_body, 0)
        return carry

    lax.fori_loop(0, nch, chunk_body, jnp.int32(0))

    @pl.when(wid < NW - 1)
    def _():
        pltpu.sync_copy(out_stage.at[pl.ds(0, NN)], out_hbm.at[pl.ds(n_lo, NN)])

    @pl.when(wid == NW - 1)
    def _():
        pltpu.sync_copy(out_stage.at[pl.ds(0, LASTN)],
                        out_hbm.at[pl.ds(n_lo, LASTN)])


_sc_agg = functools.partial(
    pl.kernel,
    out_type=jax.ShapeDtypeStruct((N, D), jnp.float32),
    mesh=plsc.VectorSubcoreMesh(core_axis_name="c", subcore_axis_name="s"),
    compiler_params=pltpu.CompilerParams(use_tc_tiling_on_sc=False),
    scratch_types=[
        pltpu.VMEM((2, CH), jnp.int32),
        pltpu.VMEM((2, CH, HD), jnp.int32),
        pltpu.VMEM((NNMAX, D), jnp.float32),
        pltpu.VMEM((WIN,), jnp.int32),
        pltpu.SMEM((WIN,), jnp.int32),
        pltpu.VMEM((WIN,), jnp.float32),
        pltpu.SMEM((WIN,), jnp.float32),
        pltpu.VMEM_SHARED((N, HD), jnp.int32),
        pltpu.SemaphoreType.DMA((2,)),
        pltpu.SemaphoreType.DMA((2,)),
        pltpu.SemaphoreType.DMA,
    ],
)(_sc_body)


def kernel(row_ptr, col_idx, node_feat, degrees):
    packed, rsd = _rowwise_call(_prologue_body, True)(node_feat, degrees)
    rp_pad = jnp.pad(row_ptr, (0, 352))
    rsd_pad = jnp.pad(rsd, (0, 352))
    return _sc_agg(rp_pad, col_idx, packed, rsd_pad)


# speedup vs baseline: 1.0066x; 1.0066x over previous
import functools

import jax
import jax.numpy as jnp
from jax import lax
from jax.experimental import pallas as pl
from jax.experimental.pallas import tpu as pltpu
from jax.experimental.pallas import tpu_sc as plsc

N = 10000
E = 320000
D = 128

NC = 2
NS = 16
NW = NC * NS
L = 16

NN = 312
LASTN = N - (NW - 1) * NN
NNMAX = max(NN, LASTN)
WIN = 352
CH = 128
KSUB = D // L
HD = D // 2
GSUB = HD // L
HMASK = -65536
BS_IT = 9
TROWS = 640
TLAST = N - (NS - 1) * TROWS


def _prologue_body(feat_ref, deg_ref, packed_ref, rsd_ref):
    rs = lax.rsqrt(deg_ref[...])
    rsd_ref[...] = rs
    scaled = feat_ref[...] * rs[:, None]
    sbf = scaled.astype(jnp.bfloat16)
    lo = lax.bitcast_convert_type(sbf[:, :HD], jnp.uint16).astype(jnp.int32)
    hi = lax.bitcast_convert_type(sbf[:, HD:], jnp.uint16).astype(jnp.int32)
    packed_ref[...] = lo | (hi << 16)


def _epilogue_body(agg_ref, rsd_ref, out_ref):
    out_ref[...] = agg_ref[...] * rsd_ref[...]


def _rowwise_call(body, packed_out):
    del packed_out
    return pl.pallas_call(
        body,
        out_shape=[
            jax.ShapeDtypeStruct((N, HD), jnp.int32),
            jax.ShapeDtypeStruct((N,), jnp.float32),
        ],
    )


def _sc_body(rp_hbm, col_hbm, feat_hbm, rsd_hbm, out_hbm,
             idx_buf, row_buf, out_stage, rp_v, rp_s, rsd_v, rsd_s, tab_sh,
             sem_idx, sem_row, sem_tab):
    cid = lax.axis_index("c")
    sid = lax.axis_index("s")
    wid = cid * NS + sid
    n_lo = wid * NN
    nn = jnp.where(wid == NW - 1, LASTN, NN).astype(jnp.int32)
    lo8 = (n_lo // 8) * 8
    off = (n_lo - lo8).astype(jnp.int32)

    t_lo = sid * TROWS

    def tab_copy(rows):
        return pltpu.make_async_copy(
            feat_hbm.at[pl.ds(t_lo, rows)], tab_sh.at[pl.ds(t_lo, rows)],
            sem_tab)

    @pl.when(sid < NS - 1)
    def _():
        tab_copy(TROWS).start()

    @pl.when(sid == NS - 1)
    def _():
        tab_copy(TLAST).start()

    pltpu.sync_copy(rp_hbm.at[pl.ds(lo8, WIN)], rp_v)
    pltpu.sync_copy(rsd_hbm.at[pl.ds(lo8, WIN)], rsd_v)
    for q in range(WIN // L):
        iv = rp_v[pl.ds(q * L, L)]
        fv = rsd_v[pl.ds(q * L, L)]
        for t in range(L):
            rp_s[q * L + t] = iv[t]
            rsd_s[q * L + t] = fv[t]

    zeros = jnp.zeros((L,), jnp.float32)

    def zbody(i, carry):
        for k in range(KSUB):
            out_stage[i, pl.ds(k * L, L)] = zeros
        return carry

    lax.fori_loop(0, NNMAX, zbody, 0)

    e0 = rp_s[off]
    eN = rp_s[off + nn]
    a0 = (e0 // 8) * 8
    nch = (eN - a0 + CH - 1) // CH

    def bsearch(base, val, hi0):
        def bb(_, s):
            lo, hi = s
            mid = (lo + hi) // 2
            v = rp_s[base + mid]
            big = v >= val
            return (jnp.where(big, lo, mid + 1), jnp.where(big, mid, hi))
        lo, _ = lax.fori_loop(0, BS_IT, bb, (jnp.int32(0), hi0))
        return lo

    def cbase(c):
        return jnp.minimum(a0 + c * CH, E - CH)

    def start_idx(c):
        pltpu.make_async_copy(
            col_hbm.at[pl.ds(cbase(c), CH)],
            idx_buf.at[c % 2], sem_idx.at[c % 2]).start()

    def wait_idx(c):
        pltpu.make_async_copy(
            col_hbm.at[pl.ds(cbase(c), CH)],
            idx_buf.at[c % 2], sem_idx.at[c % 2]).wait()

    def start_row(c):
        pltpu.make_async_copy(
            tab_sh.at[idx_buf.at[c % 2]],
            row_buf.at[c % 2], sem_row.at[c % 2]).start()

    def wait_row(c):
        pltpu.make_async_copy(
            tab_sh.at[idx_buf.at[c % 2]],
            row_buf.at[c % 2], sem_row.at[c % 2]).wait()

    @pl.when(nch > 0)
    def _():
        start_idx(0)

    @pl.when(nch > 1)
    def _():
        start_idx(1)

    @pl.when(sid < NS - 1)
    def _():
        tab_copy(TROWS).wait()

    @pl.when(sid == NS - 1)
    def _():
        tab_copy(TLAST).wait()

    plsc.subcore_barrier()

    @pl.when(nch > 0)
    def _():
        wait_idx(0)
        start_row(0)

    def chunk_body(c, carry):
        chunk_lo = a0 + c * CH
        chunk_hi = chunk_lo + CH
        base = cbase(c)
        wait_row(c)

        @pl.when(c + 2 < nch)
        def _():
            start_idx(c + 2)

        @pl.when(c + 1 < nch)
        def _():
            wait_idx(c + 1)
            start_row(c + 1)

        i_begin = bsearch(off + 1, chunk_lo + 1, nn)
        i_end = bsearch(off, chunk_hi, nn)

        def node_body(i, carry2):
            start = rp_s[off + i]
            end = rp_s[off + i + 1]
            a = jnp.maximum(start, chunk_lo)
            b = jnp.minimum(end, chunk_hi)
            accs = tuple(out_stage[i, pl.ds(k * L, L)] for k in range(KSUB))

            rb = c % 2

            def add_row(r, accs_):
                na = list(accs_)
                for g in range(GSUB):
                    w = row_buf[rb, r, pl.ds(g * L, L)]
                    lo = lax.bitcast_convert_type(w << 16, jnp.float32)
                    hi = lax.bitcast_convert_type(w & HMASK, jnp.float32)
                    na[g] = na[g] + lo
                    na[GSUB + g] = na[GSUB + g] + hi
                return tuple(na)

            r0 = a - base
            npair = (b - a) // 2

            def ebody2(p, accs_):
                r = r0 + 2 * p
                return add_row(r + 1, add_row(r, accs_))

            accs = lax.fori_loop(0, npair, ebody2, accs)

            def etail(j, accs_):
                return add_row(j - base, accs_)

            accs = lax.fori_loop(a + 2 * npair, b, etail, accs)
            m = jnp.where(end <= chunk_hi, rsd_s[off + i], 1.0)
            for k in range(KSUB):
                out_stage[i, pl.ds(k * L, L)] = accs[k] * m
            return carry2

        lax.fori_loop(i_begin, i_end, node_body, 0)
        return carry

    lax.fori_loop(0, nch, chunk_body, jnp.int32(0))

    @pl.when(wid < NW - 1)
    def _():
        pltpu.sync_copy(out_stage.at[pl.ds(0, NN)], out_hbm.at[pl.ds(n_lo, NN)])

    @pl.when(wid == NW - 1)
    def _():
        pltpu.sync_copy(out_stage.at[pl.ds(0, LASTN)],
                        out_hbm.at[pl.ds(n_lo, LASTN)])


_sc_agg = functools.partial(
    pl.kernel,
    out_type=jax.ShapeDtypeStruct((N, D), jnp.float32),
    mesh=plsc.VectorSubcoreMesh(core_axis_name="c", subcore_axis_name="s"),
    compiler_params=pltpu.CompilerParams(use_tc_tiling_on_sc=False),
    scratch_types=[
        pltpu.VMEM((2, CH), jnp.int32),
        pltpu.VMEM((2, CH, HD), jnp.int32),
        pltpu.VMEM((NNMAX, D), jnp.float32),
        pltpu.VMEM((WIN,), jnp.int32),
        pltpu.SMEM((WIN,), jnp.int32),
        pltpu.VMEM((WIN,), jnp.float32),
        pltpu.SMEM((WIN,), jnp.float32),
        pltpu.VMEM_SHARED((N, HD), jnp.int32),
        pltpu.SemaphoreType.DMA((2,)),
        pltpu.SemaphoreType.DMA((2,)),
        pltpu.SemaphoreType.DMA,
    ],
)(_sc_body)


def kernel(row_ptr, col_idx, node_feat, degrees):
    packed, rsd = _rowwise_call(_prologue_body, True)(node_feat, degrees)
    rp_pad = jnp.pad(row_ptr, (0, 352))
    rsd_pad = jnp.pad(rsd, (0, 352))
    return _sc_agg(rp_pad, col_idx, packed, rsd_pad)


# speedup vs baseline: 1.0644x; 1.0574x over previous
import functools

import jax
import jax.numpy as jnp
from jax import lax
from jax.experimental import pallas as pl
from jax.experimental.pallas import tpu as pltpu
from jax.experimental.pallas import tpu_sc as plsc

N = 10000
E = 320000
D = 128

NC = 2
NS = 16
NW = NC * NS
L = 16

NN = 312
LASTN = N - (NW - 1) * NN
NNMAX = max(NN, LASTN)
WIN = 352
CH = 128
NB = 4
KSUB = D // L
HD = D // 2
GSUB = HD // L
HMASK = -65536
BS_IT = 9
TROWS = 640
TLAST = N - (NS - 1) * TROWS


def _prologue_body(feat_ref, deg_ref, packed_ref, rsd_ref):
    rs = lax.rsqrt(deg_ref[...])
    rsd_ref[...] = rs
    scaled = feat_ref[...] * rs[:, None]
    sbf = scaled.astype(jnp.bfloat16)
    lo = lax.bitcast_convert_type(sbf[:, :HD], jnp.uint16).astype(jnp.int32)
    hi = lax.bitcast_convert_type(sbf[:, HD:], jnp.uint16).astype(jnp.int32)
    packed_ref[...] = lo | (hi << 16)


def _prologue_call(body):
    return pl.pallas_call(
        body,
        out_shape=[
            jax.ShapeDtypeStruct((N, HD), jnp.int32),
            jax.ShapeDtypeStruct((N,), jnp.float32),
        ],
    )


def _sc_body(rp_hbm, col_hbm, feat_hbm, rsd_hbm, out_hbm,
             idx_buf, row_buf, out_stage, rp_v, rp_s, rsd_v, rsd_s, tab_sh,
             sem_idx, sem_row, sem_tab):
    cid = lax.axis_index("c")
    sid = lax.axis_index("s")
    wid = cid * NS + sid
    n_lo = wid * NN
    nn = jnp.where(wid == NW - 1, LASTN, NN).astype(jnp.int32)
    lo8 = (n_lo // 8) * 8
    off = (n_lo - lo8).astype(jnp.int32)

    t_lo = sid * TROWS

    def tab_copy(rows):
        return pltpu.make_async_copy(
            feat_hbm.at[pl.ds(t_lo, rows)], tab_sh.at[pl.ds(t_lo, rows)],
            sem_tab)

    @pl.when(sid < NS - 1)
    def _():
        tab_copy(TROWS).start()

    @pl.when(sid == NS - 1)
    def _():
        tab_copy(TLAST).start()

    pltpu.sync_copy(rp_hbm.at[pl.ds(lo8, WIN)], rp_v)
    pltpu.sync_copy(rsd_hbm.at[pl.ds(lo8, WIN)], rsd_v)
    for q in range(WIN // L):
        iv = rp_v[pl.ds(q * L, L)]
        fv = rsd_v[pl.ds(q * L, L)]
        for t in range(L):
            rp_s[q * L + t] = iv[t]
            rsd_s[q * L + t] = fv[t]

    zeros = jnp.zeros((L,), jnp.float32)

    def zbody(i, carry):
        for k in range(KSUB):
            out_stage[i, pl.ds(k * L, L)] = zeros
        return carry

    lax.fori_loop(0, NNMAX, zbody, 0)

    e0 = rp_s[off]
    eN = rp_s[off + nn]
    a0 = (e0 // 8) * 8
    nch = (eN - a0 + CH - 1) // CH

    def bsearch(base, val, hi0):
        def bb(_, s):
            lo, hi = s
            mid = (lo + hi) // 2
            v = rp_s[base + mid]
            big = v >= val
            return (jnp.where(big, lo, mid + 1), jnp.where(big, mid, hi))
        lo, _ = lax.fori_loop(0, BS_IT, bb, (jnp.int32(0), hi0))
        return lo

    def cbase(c):
        return jnp.minimum(a0 + c * CH, E - CH)

    def start_idx(c):
        pltpu.make_async_copy(
            col_hbm.at[pl.ds(cbase(c), CH)],
            idx_buf.at[c % NB], sem_idx.at[c % NB]).start()

    def wait_idx(c):
        pltpu.make_async_copy(
            col_hbm.at[pl.ds(cbase(c), CH)],
            idx_buf.at[c % NB], sem_idx.at[c % NB]).wait()

    def start_row(c):
        pltpu.make_async_copy(
            tab_sh.at[idx_buf.at[c % NB]],
            row_buf.at[c % NB], sem_row.at[c % NB]).start()

    def wait_row(c):
        pltpu.make_async_copy(
            tab_sh.at[idx_buf.at[c % NB]],
            row_buf.at[c % NB], sem_row.at[c % NB]).wait()

    for cc in range(NB):
        @pl.when(nch > cc)
        def _(cc=cc):
            start_idx(cc)

    @pl.when(sid < NS - 1)
    def _():
        tab_copy(TROWS).wait()

    @pl.when(sid == NS - 1)
    def _():
        tab_copy(TLAST).wait()

    plsc.subcore_barrier()

    @pl.when(nch > 0)
    def _():
        wait_idx(0)
        start_row(0)

    @pl.when(nch > 1)
    def _():
        wait_idx(1)
        start_row(1)

    def chunk_body(c, cur):
        chunk_lo = a0 + c * CH
        chunk_hi = chunk_lo + CH
        base = cbase(c)
        wait_row(c)

        @pl.when(c + NB < nch)
        def _():
            start_idx(c + NB)

        @pl.when(c + 2 < nch)
        def _():
            wait_idx(c + 2)
            start_row(c + 2)

        i_begin = cur
        i_end = bsearch(off, chunk_hi, nn)

        def node_body(i, carry2):
            start = rp_s[off + i]
            end = rp_s[off + i + 1]
            a = jnp.maximum(start, chunk_lo)
            b = jnp.minimum(end, chunk_hi)
            accs = tuple(out_stage[i, pl.ds(k * L, L)] for k in range(KSUB))

            rb = c % NB

            def add_row(r, accs_):
                na = list(accs_)
                for g in range(GSUB):
                    w = row_buf[rb, r, pl.ds(g * L, L)]
                    lo = lax.bitcast_convert_type(w << 16, jnp.float32)
                    hi = lax.bitcast_convert_type(w & HMASK, jnp.float32)
                    na[g] = na[g] + lo
                    na[GSUB + g] = na[GSUB + g] + hi
                return tuple(na)

            r0 = a - base
            npair = (b - a) // 2

            def ebody2(p, accs_):
                r = r0 + 2 * p
                return add_row(r + 1, add_row(r, accs_))

            accs = lax.fori_loop(0, npair, ebody2, accs)

            def etail(j, accs_):
                return add_row(j - base, accs_)

            accs = lax.fori_loop(a + 2 * npair, b, etail, accs)
            m = jnp.where(end <= chunk_hi, rsd_s[off + i], 1.0)
            for k in range(KSUB):
                out_stage[i, pl.ds(k * L, L)] = accs[k] * m
            return carry2

        lax.fori_loop(i_begin, i_end, node_body, 0)
        ncur = jnp.where(rp_s[off + i_end] <= chunk_hi, i_end,
                         jnp.maximum(i_end - 1, cur))
        return ncur

    lax.fori_loop(0, nch, chunk_body, jnp.int32(0))

    @pl.when(wid < NW - 1)
    def _():
        pltpu.sync_copy(out_stage.at[pl.ds(0, NN)], out_hbm.at[pl.ds(n_lo, NN)])

    @pl.when(wid == NW - 1)
    def _():
        pltpu.sync_copy(out_stage.at[pl.ds(0, LASTN)],
                        out_hbm.at[pl.ds(n_lo, LASTN)])


_sc_agg = functools.partial(
    pl.kernel,
    out_type=jax.ShapeDtypeStruct((N, D), jnp.float32),
    mesh=plsc.VectorSubcoreMesh(core_axis_name="c", subcore_axis_name="s"),
    compiler_params=pltpu.CompilerParams(use_tc_tiling_on_sc=False),
    scratch_types=[
        pltpu.VMEM((NB, CH), jnp.int32),
        pltpu.VMEM((NB, CH, HD), jnp.int32),
        pltpu.VMEM((NNMAX, D), jnp.float32),
        pltpu.VMEM((WIN,), jnp.int32),
        pltpu.SMEM((WIN,), jnp.int32),
        pltpu.VMEM((WIN,), jnp.float32),
        pltpu.SMEM((WIN,), jnp.float32),
        pltpu.VMEM_SHARED((N, HD), jnp.int32),
        pltpu.SemaphoreType.DMA((NB,)),
        pltpu.SemaphoreType.DMA((NB,)),
        pltpu.SemaphoreType.DMA,
    ],
)(_sc_body)


def kernel(row_ptr, col_idx, node_feat, degrees):
    packed, rsd = _prologue_call(_prologue_body)(node_feat, degrees)
    rp_pad = jnp.pad(row_ptr, (0, 352))
    rsd_pad = jnp.pad(rsd, (0, 352))
    return _sc_agg(rp_pad, col_idx, packed, rsd_pad)
